# baseline (speedup 1.0000x reference)
import functools

import jax
import jax.numpy as jnp
from jax import lax
from jax.experimental import pallas as pl
from jax.experimental.pallas import tpu as pltpu
from jax.experimental.pallas import tpu_sc as plsc

DIM = 64
GAMMA = 12.0
CHUNK = 128


@functools.cache
def _make_sc_kernel(B: int):
    info = plsc.get_sparse_core_info()
    NC, NS, L = info.num_cores, info.num_subcores, info.num_lanes
    NW = NC * NS
    BW = B // NW
    NCHUNK = BW // CHUNK
    mesh = plsc.VectorSubcoreMesh(core_axis_name="c", subcore_axis_name="s")

    @functools.partial(
        pl.kernel,
        mesh=mesh,
        compiler_params=pltpu.CompilerParams(
            needs_layout_passes=False, use_tc_tiling_on_sc=False
        ),
        out_type=jax.ShapeDtypeStruct((B,), jnp.float32),
        scratch_types=[
            pltpu.VMEM((BW,), jnp.int32),
            pltpu.VMEM((BW,), jnp.int32),
            pltpu.VMEM((BW,), jnp.int32),
            pltpu.VMEM((BW, DIM), jnp.float32),
            pltpu.VMEM((BW, DIM), jnp.float32),
            pltpu.VMEM((BW, DIM), jnp.float32),
            pltpu.VMEM((BW,), jnp.float32),
            pltpu.SemaphoreType.DMA,
        ],
    )
    def k(hidx_hbm, ridx_hbm, tidx_hbm, ent_hbm, rel_hbm, out_hbm,
          hidx_v, ridx_v, tidx_v, h_v, r_v, t_v, out_v, sem):
        wid = lax.axis_index("s") * NC + lax.axis_index("c")
        base = wid * BW
        pltpu.sync_copy(hidx_hbm.at[pl.ds(base, BW)], hidx_v)
        pltpu.sync_copy(ridx_hbm.at[pl.ds(base, BW)], ridx_v)
        pltpu.sync_copy(tidx_hbm.at[pl.ds(base, BW)], tidx_v)
        copies = []
        for j in range(NCHUNK):
            src = pl.ds(j * CHUNK, CHUNK)
            copies.append(pltpu.async_copy(
                ent_hbm.at[hidx_v.at[src]], h_v.at[src], sem))
            copies.append(pltpu.async_copy(
                rel_hbm.at[ridx_v.at[src]], r_v.at[src], sem))
            copies.append(pltpu.async_copy(
                ent_hbm.at[tidx_v.at[src]], t_v.at[src], sem))
        for c in copies:
            c.wait()

        lanes = lax.iota(jnp.int32, L)

        def group(g, carry):
            def sample_step(j, vec):
                i = g * L + j
                acc = jnp.zeros((L,), jnp.float32)
                for c in range(DIM // L):
                    sl = pl.ds(c * L, L)
                    acc = acc + jnp.abs(h_v[i, sl] + r_v[i, sl] - t_v[i, sl])
                return jnp.where(lanes == j, GAMMA - jnp.sum(acc), vec)

            vec = lax.fori_loop(0, L, sample_step, jnp.zeros((L,), jnp.float32))
            out_v[pl.ds(g * L, L)] = vec
            return carry

        lax.fori_loop(0, BW // L, group, 0)
        pltpu.sync_copy(out_v, out_hbm.at[pl.ds(base, BW)])

    return k


@jax.jit
def kernel(sample, entity_embedding, relation_embedding):
    B = sample.shape[0]
    hidx = sample[:, 0]
    ridx = sample[:, 1]
    tidx = sample[:, 2]
    score = _make_sc_kernel(B)(hidx, ridx, tidx, entity_embedding, relation_embedding)
    return score.reshape(B, 1)
